# Initial kernel scaffold; baseline (speedup 1.0000x reference)
import jax
import jax.numpy as jnp
from jax.experimental import pallas as pl


def kernel(xyz, params):
    raise NotImplementedError("write your pallas kernel here")


# speedup vs baseline: 25.1214x; 25.1214x over previous
import functools
from functools import partial

import jax
import jax.numpy as jnp
import numpy as np
from jax.experimental import pallas as pl
from jax.experimental.pallas import tpu as pltpu

_B, _N, _JOINTS = 8, 1024, 3


def _fold_cbr(l):
    s = l['g'] / jnp.sqrt(l['v'] + 1e-5)
    return l['W'] * s[None, :], (l['b'] - l['m']) * s + l['be']


def _fold_lin_bn(lin, bn):
    s = bn['g'] / jnp.sqrt(bn['v'] + 1e-5)
    return lin['W'] * s[None, :], (lin['b'] - bn['m']) * s + bn['be']


def _fps_body(S, x_ref, out_ref, dist_ref, oh_ref):
    B, _, N = x_ref.shape
    iota = jax.lax.broadcasted_iota(jnp.int32, (B, N), 1)
    dist_ref[...] = jnp.full((B, N), 1e10, jnp.float32)
    oh_ref[...] = (iota == 0).astype(jnp.float32)

    def step(i, _):
        oh = oh_ref[...]
        x0 = x_ref[:, 0, :]
        x1 = x_ref[:, 1, :]
        x2 = x_ref[:, 2, :]
        c0 = jnp.sum(oh * x0, axis=1, keepdims=True)
        c1 = jnp.sum(oh * x1, axis=1, keepdims=True)
        c2 = jnp.sum(oh * x2, axis=1, keepdims=True)
        cent = jnp.concatenate(
            [c0, c1, c2, jnp.zeros((B, 5), jnp.float32)], axis=1)
        out_ref[pl.ds(i, 1), :, :] = cent[None]
        d = (x0 - c0) ** 2 + (x1 - c1) ** 2 + (x2 - c2) ** 2
        dist = jnp.minimum(dist_ref[...], d)
        dist_ref[...] = dist
        m = jnp.max(dist, axis=1, keepdims=True)
        cand = jnp.where(dist == m, iota, N)
        far = jnp.min(cand, axis=1, keepdims=True)
        oh_ref[...] = (iota == far).astype(jnp.float32)
        return 0

    jax.lax.fori_loop(0, S, step, 0)


def _fps(xpad, S):
    B, _, N = xpad.shape
    return pl.pallas_call(
        partial(_fps_body, S),
        out_shape=jax.ShapeDtypeStruct((S, B, 8), jnp.float32),
        scratch_shapes=[pltpu.VMEM((B, N), jnp.float32),
                        pltpu.VMEM((B, N), jnp.float32)],
    )(xpad)


def _proj1_body(x_ref, nx_ref, w_ref, b_ref, p_ref, c_ref):
    p_ref[0] = jnp.dot(x_ref[0], w_ref[...],
                       preferred_element_type=jnp.float32) + b_ref[...]
    c_ref[0] = jnp.dot(nx_ref[0], w_ref[...],
                       preferred_element_type=jnp.float32)


def _proj2_body(f_ref, x_ref, nx_ref, wf_ref, wx_ref, b_ref, p_ref, c_ref):
    p = jnp.dot(f_ref[0], wf_ref[...], preferred_element_type=jnp.float32)
    p = p + jnp.dot(x_ref[0], wx_ref[...], preferred_element_type=jnp.float32)
    p_ref[0] = p + b_ref[...]
    c_ref[0] = jnp.dot(nx_ref[0], wx_ref[...],
                       preferred_element_type=jnp.float32)


def _project(feats, xrow, nxrow, Wf, Wx, b):
    B, N, _ = xrow.shape
    S = nxrow.shape[1]
    H = Wx.shape[1]
    if feats is None:
        grid = (B,)
        return pl.pallas_call(
            _proj1_body,
            grid=grid,
            in_specs=[
                pl.BlockSpec((1, N, 8), lambda b_: (b_, 0, 0)),
                pl.BlockSpec((1, S, 8), lambda b_: (b_, 0, 0)),
                pl.BlockSpec((8, H), lambda b_: (0, 0)),
                pl.BlockSpec((1, H), lambda b_: (0, 0)),
            ],
            out_specs=[
                pl.BlockSpec((1, N, H), lambda b_: (b_, 0, 0)),
                pl.BlockSpec((1, S, H), lambda b_: (b_, 0, 0)),
            ],
            out_shape=[
                jax.ShapeDtypeStruct((B, N, H), jnp.float32),
                jax.ShapeDtypeStruct((B, S, H), jnp.float32),
            ],
        )(xrow, nxrow, Wx, b)
    Cf = feats.shape[2]
    return pl.pallas_call(
        _proj2_body,
        grid=(B,),
        in_specs=[
            pl.BlockSpec((1, N, Cf), lambda b_: (b_, 0, 0)),
            pl.BlockSpec((1, N, 8), lambda b_: (b_, 0, 0)),
            pl.BlockSpec((1, S, 8), lambda b_: (b_, 0, 0)),
            pl.BlockSpec((Cf, H), lambda b_: (0, 0)),
            pl.BlockSpec((8, H), lambda b_: (0, 0)),
            pl.BlockSpec((1, H), lambda b_: (0, 0)),
        ],
        out_specs=[
            pl.BlockSpec((1, N, H), lambda b_: (b_, 0, 0)),
            pl.BlockSpec((1, S, H), lambda b_: (b_, 0, 0)),
        ],
        out_shape=[
            jax.ShapeDtypeStruct((B, N, H), jnp.float32),
            jax.ShapeDtypeStruct((B, S, H), jnp.float32),
        ],
    )(feats, xrow, nxrow, Wf, Wx, b)


def _branch_body(r2, K, Sb, nx_ref, xt_ref, p_ref, cp_ref, t_ref,
                 w2_ref, b2_ref, w3_ref, b3_ref, out_ref):
    N = xt_ref.shape[2]
    H1 = p_ref.shape[2]
    nx = nx_ref[0]
    xt = xt_ref[0]
    d = (jnp.sum(nx * nx, axis=1, keepdims=True)
         + jnp.sum(xt * xt, axis=0, keepdims=True)
         - 2.0 * jnp.dot(nx, xt, preferred_element_type=jnp.float32))
    maskf = (d <= r2).astype(jnp.float32)
    rank_i = jnp.dot(maskf, t_ref[...], preferred_element_type=jnp.float32)
    rank_e = rank_i - maskf
    count = jnp.sum(maskf, axis=1, keepdims=True).astype(jnp.int32)
    rank_ei = rank_e.astype(jnp.int32)
    k_iota = jax.lax.broadcasted_iota(jnp.int32, (Sb, K, N), 1)
    sel = (rank_ei.reshape(Sb, 1, N) == k_iota) & (
        maskf.reshape(Sb, 1, N) > 0.0)
    G = sel.astype(jnp.float32).reshape(Sb * K, N)
    g = jnp.dot(G, p_ref[0], preferred_element_type=jnp.float32)
    cp = jnp.broadcast_to(cp_ref[0].reshape(Sb, 1, H1),
                          (Sb, K, H1)).reshape(Sb * K, H1)
    z = jax.nn.relu(g - cp)
    z = jax.nn.relu(jnp.dot(z, w2_ref[...],
                            preferred_element_type=jnp.float32) + b2_ref[...])
    z = jax.nn.relu(jnp.dot(z, w3_ref[...],
                            preferred_element_type=jnp.float32) + b3_ref[...])
    H3 = z.shape[1]
    z = z.reshape(Sb, K, H3)
    km = jax.lax.broadcasted_iota(jnp.int32, (Sb, K, 1), 1) < jnp.minimum(
        count, K).reshape(Sb, 1, 1)
    out_ref[0] = jnp.max(jnp.where(km, z, 0.0), axis=1)


def _branch(nxrow, xt, P, C, tri, W2, b2, W3, b3, r2, K, Sb):
    B, S, _ = nxrow.shape
    N = xt.shape[2]
    H1 = P.shape[2]
    H2 = W2.shape[1]
    H3 = W3.shape[1]
    grid = (B, S // Sb)
    return pl.pallas_call(
        partial(_branch_body, r2, K, Sb),
        grid=grid,
        in_specs=[
            pl.BlockSpec((1, Sb, 8), lambda b_, s_: (b_, s_, 0)),
            pl.BlockSpec((1, 8, N), lambda b_, s_: (b_, 0, 0)),
            pl.BlockSpec((1, N, H1), lambda b_, s_: (b_, 0, 0)),
            pl.BlockSpec((1, Sb, H1), lambda b_, s_: (b_, s_, 0)),
            pl.BlockSpec((N, N), lambda b_, s_: (0, 0)),
            pl.BlockSpec((H1, H2), lambda b_, s_: (0, 0)),
            pl.BlockSpec((1, H2), lambda b_, s_: (0, 0)),
            pl.BlockSpec((H2, H3), lambda b_, s_: (0, 0)),
            pl.BlockSpec((1, H3), lambda b_, s_: (0, 0)),
        ],
        out_specs=pl.BlockSpec((1, Sb, H3), lambda b_, s_: (b_, s_, 0)),
        out_shape=jax.ShapeDtypeStruct((B, S, H3), jnp.float32),
    )(nxrow, xt, P, C, tri, W2, b2, W3, b3)


def _sa3_body(g_ref, w1_ref, b1_ref, w2_ref, b2_ref, w3_ref, b3_ref, out_ref):
    z = jax.nn.relu(jnp.dot(g_ref[0], w1_ref[...],
                            preferred_element_type=jnp.float32) + b1_ref[...])
    z = jax.nn.relu(jnp.dot(z, w2_ref[...],
                            preferred_element_type=jnp.float32) + b2_ref[...])
    z = jax.nn.relu(jnp.dot(z, w3_ref[...],
                            preferred_element_type=jnp.float32) + b3_ref[...])
    out_ref[0] = jnp.max(z, axis=0, keepdims=True)


def _sa3(g, W1, b1, W2, b2, W3, b3):
    B, S, Cin = g.shape
    H1, H2, H3 = W1.shape[1], W2.shape[1], W3.shape[1]
    return pl.pallas_call(
        _sa3_body,
        grid=(B,),
        in_specs=[
            pl.BlockSpec((1, S, Cin), lambda b_: (b_, 0, 0)),
            pl.BlockSpec((Cin, H1), lambda b_: (0, 0)),
            pl.BlockSpec((1, H1), lambda b_: (0, 0)),
            pl.BlockSpec((H1, H2), lambda b_: (0, 0)),
            pl.BlockSpec((1, H2), lambda b_: (0, 0)),
            pl.BlockSpec((H2, H3), lambda b_: (0, 0)),
            pl.BlockSpec((1, H3), lambda b_: (0, 0)),
        ],
        out_specs=pl.BlockSpec((1, 1, H3), lambda b_: (b_, 0, 0)),
        out_shape=jax.ShapeDtypeStruct((B, 1, H3), jnp.float32),
    )(g, W1, b1, W2, b2, W3, b3)[:, 0, :]


def _fc_body(x_ref, w1_ref, b1_ref, w2_ref, b2_ref, wt_ref, bt_ref,
             wa_ref, ba_ref, trs_ref, axes_ref):
    z = jax.nn.relu(jnp.dot(x_ref[...], w1_ref[...],
                            preferred_element_type=jnp.float32) + b1_ref[...])
    z = jax.nn.relu(jnp.dot(z, w2_ref[...],
                            preferred_element_type=jnp.float32) + b2_ref[...])
    trs_ref[...] = jnp.dot(z, wt_ref[...],
                           preferred_element_type=jnp.float32) + bt_ref[...]
    a = jnp.dot(z, wa_ref[...], preferred_element_type=jnp.float32) + ba_ref[...]
    tp = (1.0 / (1.0 + jnp.exp(-a))) * (2.0 * np.pi)
    t = tp[:, 0:3]
    ph = tp[:, 3:6]
    st, ct = jnp.sin(t), jnp.cos(t)
    sp, cp = jnp.sin(ph), jnp.cos(ph)
    axes_ref[...] = jnp.concatenate([st * cp, st * sp, ct], axis=1)


def _fc(x, W1, b1, W2, b2, Wt, bt, Wa, ba):
    B = x.shape[0]
    return pl.pallas_call(
        _fc_body,
        out_shape=[jax.ShapeDtypeStruct((B, 3 * _JOINTS), jnp.float32),
                   jax.ShapeDtypeStruct((B, 3 * _JOINTS), jnp.float32)],
    )(x, W1, b1, W2, b2, Wt, bt, Wa, ba)


_SA1 = dict(S=512, radii=(0.1, 0.2, 0.4), ks=(16, 32, 128), sbs=(64, 32, 8))
_SA2 = dict(S=128, radii=(0.2, 0.4, 0.8), ks=(32, 64, 128), sbs=(32, 16, 8))


def _sa_level(xpad_t, xrow, feats, S, radii, ks, sbs, branches, tri):
    B = xpad_t.shape[0]
    cent = _fps(xpad_t, S)
    nxrow = jnp.transpose(cent, (1, 0, 2))

    w1s, b1s, offs = [], [], [0]
    for layers in branches:
        W, b = _fold_cbr(layers[0])
        w1s.append(W)
        b1s.append(b)
        offs.append(offs[-1] + W.shape[1])
    Wcat = jnp.concatenate(w1s, axis=1)
    bcat = jnp.concatenate(b1s)[None, :]
    Cin = Wcat.shape[0]
    Wx = Wcat[Cin - 3:, :]
    Wxp = jnp.concatenate([Wx, jnp.zeros((5, Wx.shape[1]), jnp.float32)], 0)
    if feats is None:
        P, C = _project(None, xrow, nxrow, None, Wxp, bcat)
    else:
        Wf = Wcat[:Cin - 3, :]
        P, C = _project(feats, xrow, nxrow, Wf, Wxp, bcat)

    outs = []
    for i, layers in enumerate(branches):
        W2, b2 = _fold_cbr(layers[1])
        W3, b3 = _fold_cbr(layers[2])
        o = _branch(nxrow, xpad_t, P[:, :, offs[i]:offs[i + 1]],
                    C[:, :, offs[i]:offs[i + 1]], tri,
                    W2, b2[None, :], W3, b3[None, :],
                    radii[i] ** 2, ks[i], sbs[i])
        outs.append(o)
    return cent, nxrow, jnp.concatenate(outs, axis=-1)


def kernel(xyz, params):
    B, _, N = xyz.shape
    xpad1 = jnp.concatenate(
        [xyz, jnp.zeros((B, 5, N), jnp.float32)], axis=1)
    xrow1 = jnp.transpose(xpad1, (0, 2, 1))
    tri1 = (jax.lax.broadcasted_iota(jnp.int32, (N, N), 0)
            <= jax.lax.broadcasted_iota(jnp.int32, (N, N), 1)
            ).astype(jnp.float32)

    cent1, nx1row, l1_feat = _sa_level(
        xpad1, xrow1, None, _SA1['S'], _SA1['radii'], _SA1['ks'],
        _SA1['sbs'], params['sa1'], tri1)

    xpad2 = jnp.transpose(cent1, (1, 2, 0))
    S1 = _SA1['S']
    tri2 = tri1[:S1, :S1]
    cent2, nx2row, l2_feat = _sa_level(
        xpad2, nx1row, l1_feat, _SA2['S'], _SA2['radii'], _SA2['ks'],
        _SA2['sbs'], params['sa2'], tri2)

    l2_xyz = nx2row[:, :, :3]
    g = jnp.concatenate([l2_xyz, l2_feat], axis=-1)
    pad = (-g.shape[2]) % 8
    gpad = jnp.concatenate(
        [g, jnp.zeros((B, g.shape[1], pad), jnp.float32)], axis=2)
    sa3 = params['sa3']
    Ws, bs = zip(*[_fold_cbr(l) for l in sa3])
    W1 = jnp.concatenate([Ws[0], jnp.zeros((pad, Ws[0].shape[1]),
                                           jnp.float32)], 0)
    x = _sa3(gpad, W1, bs[0][None], Ws[1], bs[1][None], Ws[2], bs[2][None])

    Wf1, bf1 = _fold_lin_bn(params['fc1'], params['bn1'])
    Wf2, bf2 = _fold_lin_bn(params['fc2'], params['bn2'])
    Wa = params['fc3_axis']['W']
    ba = params['fc3_axis']['b']
    perm = np.array([0, 2, 4, 1, 3, 5])
    trs, axes_cat = _fc(x, Wf1, bf1[None], Wf2, bf2[None],
                        params['fc3_tr']['W'], params['fc3_tr']['b'][None],
                        Wa[:, perm], ba[perm][None])
    axes = jnp.transpose(axes_cat.reshape(B, 3, _JOINTS),
                         (0, 2, 1)).reshape(B, 3 * _JOINTS)
    l3_points = x[:, :, None]
    return (trs, axes, l3_points)
